# Initial kernel scaffold; baseline (speedup 1.0000x reference)
import jax
import jax.numpy as jnp
from jax.experimental import pallas as pl


def kernel(packed_input_ids, cu_seqlens, wte, wpe):
    raise NotImplementedError("write your pallas kernel here")


# speedup vs baseline: 1.9275x; 1.9275x over previous
import functools

import jax
import jax.numpy as jnp
from jax import lax
from jax.experimental import pallas as pl
from jax.experimental.pallas import tpu as pltpu
from jax.experimental.pallas import tpu_sc as plsc

VOCAB = 100000
N_POS = 8192
D = 1024
B = 16
TOTAL = 32768

NC = 2
NS = 16
L = 16
NW = NC * NS
TOK_W = TOTAL // NW
CH = 16
NCHUNK = TOK_W // CH
IDX_ROWS = TOK_W // L


def _body(ids_hbm, cu_hbm, wte_hbm, wpe_hbm, out_hbm,
          idx_v, pos_v, cu_v, a0, a1, b0, b1, sg0, sg1, so0, so1):
  cid = lax.axis_index("c")
  sid = lax.axis_index("s")
  wid = sid * NC + cid
  tokbase = wid * TOK_W

  pltpu.sync_copy(ids_hbm.at[pl.ds(wid * IDX_ROWS, IDX_ROWS)], idx_v)
  pltpu.sync_copy(cu_hbm, cu_v)

  cuvec = cu_v[:]
  cbs = [cuvec.at[jnp.full((L,), j, jnp.int32)].get(mode="promise_in_bounds")
         for j in range(1, B)]
  iota = lax.iota(jnp.int32, L)

  def pos_body(i, carry):
    tok = tokbase + i * L + iota
    m = jnp.zeros((L,), jnp.int32)
    for cb in cbs:
      m = jnp.maximum(m, jnp.where(cb <= tok, cb, jnp.int32(0)))
    pos_v[i, :] = tok - m
    return carry

  lax.fori_loop(0, IDX_ROWS, pos_body, 0)

  def start_gather(ch, a, b, sg):
    pltpu.make_async_copy(wte_hbm.at[idx_v.at[ch]], a, sg).start()
    pltpu.make_async_copy(wpe_hbm.at[pos_v.at[ch]], b, sg).start()

  def wait_gather(a, b, sg):
    pltpu.make_async_copy(wte_hbm.at[pl.ds(0, CH)], a, sg).wait()
    pltpu.make_async_copy(wte_hbm.at[pl.ds(0, CH)], b, sg).wait()

  def do_add(a, b):
    def add_body(k, carry):
      for r in range(CH):
        sl = pl.ds(k * L, L)
        a[r, sl] = a[r, sl] + b[r, sl]
      return carry
    lax.fori_loop(0, D // L, add_body, 0)

  def start_store(ch, a, so):
    dst = out_hbm.at[pl.ds(tokbase + ch * CH, CH)]
    pltpu.make_async_copy(a, dst, so).start()

  def wait_store(a, so):
    pltpu.make_async_copy(a, out_hbm.at[pl.ds(0, CH)], so).wait()

  bufs = ((a0, b0, sg0, so0), (a1, b1, sg1, so1))

  start_gather(0, a0, b0, sg0)
  start_gather(1, a1, b1, sg1)
  wait_gather(a0, b0, sg0)
  do_add(a0, b0)
  start_store(0, a0, so0)

  def main_body(j, carry):
    for ph in (1, 0):
      ch = 2 * j + 1 + (1 - ph)
      a, b, sg, so = bufs[ph]
      an, bn, sgn, son = bufs[1 - ph]
      wait_store(an, son)
      start_gather(ch + 1, an, bn, sgn)
      wait_gather(a, b, sg)
      do_add(a, b)
      start_store(ch, a, so)
    return carry

  lax.fori_loop(0, (NCHUNK - 2) // 2, main_body, 0)

  wait_store(a0, so0)
  wait_gather(a1, b1, sg1)
  do_add(a1, b1)
  start_store(NCHUNK - 1, a1, so1)
  wait_store(a1, so1)


@functools.partial(jax.jit, static_argnames=())
def kernel(packed_input_ids, cu_seqlens, wte, wpe):
  ids2d = packed_input_ids.reshape(TOTAL // L, L)
  cu16 = cu_seqlens[:B].astype(jnp.int32)
  mesh = plsc.VectorSubcoreMesh(core_axis_name="c", subcore_axis_name="s")
  k = pl.kernel(
      _body,
      out_type=jax.ShapeDtypeStruct((TOTAL, D), jnp.float32),
      mesh=mesh,
      scratch_types=[
          pltpu.VMEM((IDX_ROWS, L), jnp.int32),
          pltpu.VMEM((IDX_ROWS, L), jnp.int32),
          pltpu.VMEM((B,), jnp.int32),
          pltpu.VMEM((CH, D), jnp.float32),
          pltpu.VMEM((CH, D), jnp.float32),
          pltpu.VMEM((CH, D), jnp.float32),
          pltpu.VMEM((CH, D), jnp.float32),
          pltpu.SemaphoreType.DMA,
          pltpu.SemaphoreType.DMA,
          pltpu.SemaphoreType.DMA,
          pltpu.SemaphoreType.DMA,
      ],
  )
  return k(ids2d, cu16, wte, wpe)
